# Initial kernel scaffold; baseline (speedup 1.0000x reference)
import jax
import jax.numpy as jnp
from jax.experimental import pallas as pl


def kernel(feat, edge_index, etypes, weight):
    raise NotImplementedError("write your pallas kernel here")


# speedup vs baseline: 16.9224x; 16.9224x over previous
import functools

import jax
import jax.numpy as jnp
from jax import lax
from jax.experimental import pallas as pl
from jax.experimental.pallas import tpu as pltpu
from jax.experimental.pallas import tpu_sc as plsc

_CHUNK = 128
_NUM_CORES = 2
_NUM_SUBCORES = 16
_NW = _NUM_CORES * _NUM_SUBCORES


def _mm_body(f_ref, w_ref, o_ref):
    o_ref[0] = jnp.dot(f_ref[...], w_ref[0], preferred_element_type=jnp.float32)


def _transform_nodes(feat, weight):
    n, f = feat.shape
    r, _, o = weight.shape
    bn = 2000
    return pl.pallas_call(
        _mm_body,
        grid=(n // bn, r),
        in_specs=[
            pl.BlockSpec((bn, f), lambda i, j: (i, 0)),
            pl.BlockSpec((1, f, o), lambda i, j: (j, 0, 0)),
        ],
        out_specs=pl.BlockSpec((1, bn, o), lambda i, j: (j, i, 0)),
        out_shape=jax.ShapeDtypeStruct((r, n, o), jnp.float32),
    )(feat, weight)


def _add_body(p_ref, o_ref):
    o_ref[...] = p_ref[0] + p_ref[1]


def _combine_partials(partials):
    _, n, o = partials.shape
    bn = 2000
    return pl.pallas_call(
        _add_body,
        grid=(n // bn,),
        in_specs=[pl.BlockSpec((2, bn, o), lambda i: (0, i, 0))],
        out_specs=pl.BlockSpec((bn, o), lambda i: (i, 0)),
        out_shape=jax.ShapeDtypeStruct((n, o), jnp.float32),
    )(partials)


def _make_sc_edge_kernel(n, o, e, num_rels):
    del num_rels
    nchunks_total = e // _CHUNK
    rows_per_tile = (n // (_NUM_SUBCORES * 8)) * 8
    tail_rows = n - _NUM_SUBCORES * rows_per_tile
    tail_row0 = _NUM_SUBCORES * rows_per_tile
    mesh = plsc.VectorSubcoreMesh(
        core_axis_name="c", subcore_axis_name="s",
        num_cores=_NUM_CORES, num_subcores=_NUM_SUBCORES)

    @functools.partial(
        pl.kernel,
        out_type=jax.ShapeDtypeStruct((_NUM_CORES, n, o), jnp.float32),
        mesh=mesh,
        scratch_types=[
            pltpu.VMEM((_CHUNK,), jnp.int32),
            pltpu.VMEM((_CHUNK,), jnp.int32),
            pltpu.VMEM((_CHUNK,), jnp.int32),
            pltpu.VMEM((_CHUNK,), jnp.int32),
            pltpu.VMEM((_CHUNK, o), jnp.float32),
            pltpu.VMEM_SHARED((n, o), jnp.float32),
            pltpu.SemaphoreType.DMA,
        ],
    )
    def sc_edge(t2_hbm, et_hbm, src_hbm, dst_hbm, z_hbm, out_hbm,
                et_v, src_v, dst_v, idx_v, rows_v, acc, sem):
        c = lax.axis_index("c")
        s = lax.axis_index("s")
        w = s * _NUM_CORES + c

        row0 = s * rows_per_tile
        pltpu.sync_copy(z_hbm.at[pl.ds(row0, rows_per_tile), :],
                        acc.at[pl.ds(row0, rows_per_tile), :])
        if tail_rows:
            @pl.when(s == _NUM_SUBCORES - 1)
            def _():
                pltpu.sync_copy(z_hbm.at[pl.ds(tail_row0, tail_rows), :],
                                acc.at[pl.ds(tail_row0, tail_rows), :])
        plsc.subcore_barrier()

        nk = (nchunks_total - w + _NW - 1) // _NW

        def body(k, carry):
            base = (w + k * _NW) * _CHUNK
            pltpu.sync_copy(et_hbm.at[pl.ds(base, _CHUNK)], et_v)
            pltpu.sync_copy(src_hbm.at[pl.ds(base, _CHUNK)], src_v)
            pltpu.sync_copy(dst_hbm.at[pl.ds(base, _CHUNK)], dst_v)
            for j in range(_CHUNK // 16):
                sl = pl.ds(j * 16, 16)
                idx_v[sl] = et_v[sl] * n + src_v[sl]
            pltpu.async_copy(t2_hbm.at[idx_v], rows_v, sem).wait()
            pltpu.sync_copy(rows_v, acc.at[dst_v], add=True)
            return carry

        lax.fori_loop(0, nk, body, 0)
        plsc.subcore_barrier()
        pltpu.sync_copy(acc.at[pl.ds(row0, rows_per_tile), :],
                        out_hbm.at[c, pl.ds(row0, rows_per_tile), :])
        if tail_rows:
            @pl.when(s == _NUM_SUBCORES - 1)
            def _():
                pltpu.sync_copy(acc.at[pl.ds(tail_row0, tail_rows), :],
                                out_hbm.at[c, pl.ds(tail_row0, tail_rows), :])

    return sc_edge


def kernel(feat, edge_index, etypes, weight):
    n, f = feat.shape
    num_rels, _, o = weight.shape
    e = etypes.shape[0]
    src = edge_index[0]
    dst = edge_index[1]

    t = _transform_nodes(feat, weight)
    t2 = t.reshape(num_rels * n, o)
    zeros = jnp.zeros((n, o), jnp.float32)
    sc_edge = _make_sc_edge_kernel(n, o, e, num_rels)
    partials = sc_edge(t2, etypes, src, dst, zeros)
    return _combine_partials(partials)
